# baseline (speedup 1.0000x reference)
import functools
import math

import jax
import jax.numpy as jnp
import numpy as np
from jax import lax
from jax.experimental import pallas as pl
from jax.experimental.pallas import tpu as pltpu
from jax.experimental.pallas import tpu_sc as plsc

_B = 128
_D_IN = 2048
_F = 4096
_QUANTILE = 0.7

_NC, _NS, _LANES = 2, 16, 16
_NW = _NC * _NS
_SLICE = _F // _NW

_INT_MAX = np.int32(2147483647)
_INT_MIN = np.int32(-2147483648)

_F1 = 1024
_F2 = 512
_N1 = _F // _F1
_N2 = _F // _F2
_P1 = _N1
_P2 = _N1 + _N2
_NSTEPS = _P2 + _N2


def _sc_indicator_body(sel_hbm, out_hbm, sel_v, slice_v):
    wid = lax.axis_index("s") * _NC + lax.axis_index("c")
    base = wid * _SLICE
    k_total = sel_hbm.shape[0]
    pltpu.sync_copy(sel_hbm, sel_v)

    zeros16 = jnp.zeros((_LANES,), jnp.int32)

    def _zero(i, carry):
        slice_v[pl.ds(i * _LANES, _LANES)] = zeros16
        return carry

    lax.fori_loop(0, _SLICE // _LANES, _zero, 0)

    ones16 = jnp.ones((_LANES,), jnp.int32)

    def _scatter(j, carry):
        idx = sel_v[pl.ds(j * _LANES, _LANES)]
        loc = idx - base
        m = (loc >= 0) & (loc < _SLICE)
        locc = jnp.clip(loc, 0, _SLICE - 1)
        plsc.store_scatter(slice_v, [locc], ones16, mask=m)
        return carry

    lax.fori_loop(0, k_total // _LANES, _scatter, 0)
    pltpu.sync_copy(slice_v, out_hbm.at[pl.ds(base, _SLICE)])


def _build_indicator(sel_idx):
    kern = pl.kernel(
        _sc_indicator_body,
        out_type=jax.ShapeDtypeStruct((_F,), jnp.int32),
        mesh=plsc.VectorSubcoreMesh(
            core_axis_name="c", subcore_axis_name="s",
            num_cores=_NC, num_subcores=_NS),
        scratch_types=[
            pltpu.VMEM((sel_idx.shape[0],), jnp.int32),
            pltpu.VMEM((_SLICE,), jnp.int32),
        ],
        compiler_params=pltpu.CompilerParams(needs_layout_passes=False),
    )
    return kern(sel_idx)


def _mega_body(kk, x_ref, wo_ref, bo_ref, wp_ref, bp_ref, sel_ref, gt_ref,
               o_ref, y_s, keys_s, thr_s):
    j = pl.program_id(0)

    @pl.when(j < _P1)
    def _mm1():
        col = pl.multiple_of(j * _F1, _F1)
        y_s[:, pl.ds(col, _F1)] = (
            jnp.dot(x_ref[...], wo_ref[...], preferred_element_type=jnp.float32)
            + bo_ref[...]
        )

    @pl.when((j >= _P1) & (j < _P2))
    def _mm2():
        col = pl.multiple_of((j - _P1) * _F2, _F2)
        z = (
            jnp.dot(y_s[...], wp_ref[...], preferred_element_type=jnp.float32)
            + bp_ref[...]
        )
        bits = lax.bitcast_convert_type(z, jnp.int32)
        key = bits ^ ((bits >> 31) & np.int32(0x7FFFFFFF))
        keys_s[:, pl.ds(col, _F2)] = jnp.where(sel_ref[...] != 0, key, _INT_MAX)

    @pl.when(j == _P2)
    def _bisect():
        keys = keys_s[...]
        ones = jnp.ones((_F, 128), jnp.bfloat16)
        kkf = jnp.float32(kk)

        def _count_ge(mid):
            cmp = (keys <= mid).astype(jnp.bfloat16)
            cnt = jnp.dot(cmp, ones, preferred_element_type=jnp.float32)
            return cnt[:, :1] >= kkf

        def _cond(lohi):
            lo, hi = lohi
            return jnp.any(lo < hi)

        def _it(lohi):
            lo, hi = lohi
            mid = (lo & hi) + ((lo ^ hi) >> 1)
            ge = _count_ge(mid)
            return jnp.where(ge, lo, mid + 1), jnp.where(ge, mid, hi)

        lo0 = jnp.min(keys, axis=1, keepdims=True)
        hi0 = jnp.max(jnp.where(keys == _INT_MAX, _INT_MIN, keys),
                      axis=1, keepdims=True)
        _, thr = lax.while_loop(_cond, _it, (lo0, hi0))
        thr_s[...] = thr

    @pl.when(j >= _P2)
    def _merge():
        col = pl.multiple_of((j - _P2) * _F2, _F2)
        keys_blk = keys_s[:, pl.ds(col, _F2)]
        o_ref[...] = jnp.where(keys_blk <= thr_s[...],
                               gt_ref[...], y_s[:, pl.ds(col, _F2)])


def kernel(x, W_orig, b_orig, W_policy, b_policy, ground_truth, sel_idx):
    K = sel_idx.shape[0]
    kk = int(max(1, min(K, 1 + math.floor(_QUANTILE * (K - 1)))))

    is_sel = _build_indicator(sel_idx).reshape(1, _F)
    b_o = b_orig.reshape(1, _F)
    b_p = b_policy.reshape(1, _F)

    out = pl.pallas_call(
        functools.partial(_mega_body, kk),
        grid=(_NSTEPS,),
        in_specs=[
            pl.BlockSpec((_B, _D_IN), lambda j: (0, 0)),
            pl.BlockSpec((_D_IN, _F1), lambda j: (0, jnp.minimum(j, _N1 - 1))),
            pl.BlockSpec((1, _F1), lambda j: (0, jnp.minimum(j, _N1 - 1))),
            pl.BlockSpec((_F, _F2),
                         lambda j: (0, jnp.clip(j - _P1, 0, _N2 - 1))),
            pl.BlockSpec((1, _F2),
                         lambda j: (0, jnp.clip(j - _P1, 0, _N2 - 1))),
            pl.BlockSpec((1, _F2),
                         lambda j: (0, jnp.clip(j - _P1, 0, _N2 - 1))),
            pl.BlockSpec((_B, _F2),
                         lambda j: (0, jnp.clip(j - _P2, 0, _N2 - 1))),
        ],
        out_specs=pl.BlockSpec((_B, _F2),
                               lambda j: (0, jnp.clip(j - _P2, 0, _N2 - 1))),
        out_shape=jax.ShapeDtypeStruct((_B, _F), jnp.float32),
        scratch_shapes=[
            pltpu.VMEM((_B, _F), jnp.float32),
            pltpu.VMEM((_B, _F), jnp.int32),
            pltpu.VMEM((_B, 1), jnp.int32),
        ],
        compiler_params=pltpu.CompilerParams(
            dimension_semantics=("arbitrary",)),
    )(x, W_orig, b_o, W_policy, b_p, is_sel, ground_truth)
    return out


# speedup vs baseline: 1.1406x; 1.1406x over previous
import functools
import math

import jax
import jax.numpy as jnp
import numpy as np
from jax import lax
from jax.experimental import pallas as pl
from jax.experimental.pallas import tpu as pltpu
from jax.experimental.pallas import tpu_sc as plsc

_B = 128
_D_IN = 2048
_F = 4096
_QUANTILE = 0.7

_NC, _NS, _LANES = 2, 16, 16
_NW = _NC * _NS
_SLICE = _F // _NW

_INT_MAX = np.int32(2147483647)
_INT_MIN = np.int32(-2147483648)

_F1 = 1024
_F2 = 512
_N1 = _F // _F1
_N2 = _F // _F2
_P1 = _N1
_P2 = _N1 + _N2
_NSTEPS = _P2 + _N2


def _sc_indicator_body(sel_hbm, out_hbm, sel_v, slice_v):
    wid = lax.axis_index("s") * _NC + lax.axis_index("c")
    base = wid * _SLICE
    k_total = sel_hbm.shape[0]
    pltpu.sync_copy(sel_hbm, sel_v)

    zeros16 = jnp.zeros((_LANES,), jnp.int32)

    def _zero(i, carry):
        slice_v[pl.ds(i * _LANES, _LANES)] = zeros16
        return carry

    lax.fori_loop(0, _SLICE // _LANES, _zero, 0)

    ones16 = jnp.ones((_LANES,), jnp.int32)

    def _scatter(j, carry):
        idx = sel_v[pl.ds(j * _LANES, _LANES)]
        loc = idx - base
        m = (loc >= 0) & (loc < _SLICE)
        locc = jnp.clip(loc, 0, _SLICE - 1)
        plsc.store_scatter(slice_v, [locc], ones16, mask=m)
        return carry

    lax.fori_loop(0, k_total // _LANES, _scatter, 0)
    pltpu.sync_copy(slice_v, out_hbm.at[pl.ds(base, _SLICE)])


def _build_indicator(sel_idx):
    kern = pl.kernel(
        _sc_indicator_body,
        out_type=jax.ShapeDtypeStruct((_F,), jnp.int32),
        mesh=plsc.VectorSubcoreMesh(
            core_axis_name="c", subcore_axis_name="s",
            num_cores=_NC, num_subcores=_NS),
        scratch_types=[
            pltpu.VMEM((sel_idx.shape[0],), jnp.int32),
            pltpu.VMEM((_SLICE,), jnp.int32),
        ],
        compiler_params=pltpu.CompilerParams(needs_layout_passes=False),
    )
    return kern(sel_idx)


def _mega_body(kk, x_ref, wo_ref, bo_ref, wp_ref, bp_ref, sel_ref, gt_ref,
               o_ref, y_s, keys_s, thr_s):
    j = pl.program_id(0)

    @pl.when(j < _P1)
    def _mm1():
        col = pl.multiple_of(j * _F1, _F1)
        y_s[:, pl.ds(col, _F1)] = (
            jnp.dot(x_ref[...], wo_ref[...], preferred_element_type=jnp.float32)
            + bo_ref[...]
        )

    @pl.when((j >= _P1) & (j < _P2))
    def _mm2():
        col = pl.multiple_of((j - _P1) * _F2, _F2)
        z = (
            jnp.dot(y_s[...], wp_ref[...], preferred_element_type=jnp.float32)
            + bp_ref[...]
        )
        bits = lax.bitcast_convert_type(z, jnp.int32)
        key = bits ^ ((bits >> 31) & np.int32(0x7FFFFFFF))
        keys_s[:, pl.ds(col, _F2)] = jnp.where(sel_ref[...] != 0, key, _INT_MAX)

    @pl.when(j == _P2)
    def _bisect():
        keys = keys_s[...]

        def _it(_, lohi):
            lo, hi = lohi
            mid = (lo & hi) + ((lo ^ hi) >> 1)
            cnt = jnp.sum((keys <= mid).astype(jnp.int32), axis=1,
                          keepdims=True)
            ge = cnt >= kk
            return jnp.where(ge, lo, mid + 1), jnp.where(ge, mid, hi)

        lo0 = jnp.full((_B, 1), _INT_MIN, jnp.int32)
        hi0 = jnp.full((_B, 1), np.int32(0x7F800000), jnp.int32)
        _, thr = lax.fori_loop(0, 32, _it, (lo0, hi0))
        thr_s[...] = thr

    @pl.when(j >= _P2)
    def _merge():
        col = pl.multiple_of((j - _P2) * _F2, _F2)
        keys_blk = keys_s[:, pl.ds(col, _F2)]
        o_ref[...] = jnp.where(keys_blk <= thr_s[...],
                               gt_ref[...], y_s[:, pl.ds(col, _F2)])


def kernel(x, W_orig, b_orig, W_policy, b_policy, ground_truth, sel_idx):
    K = sel_idx.shape[0]
    kk = int(max(1, min(K, 1 + math.floor(_QUANTILE * (K - 1)))))

    is_sel = _build_indicator(sel_idx).reshape(1, _F)
    b_o = b_orig.reshape(1, _F)
    b_p = b_policy.reshape(1, _F)

    out = pl.pallas_call(
        functools.partial(_mega_body, kk),
        grid=(_NSTEPS,),
        in_specs=[
            pl.BlockSpec((_B, _D_IN), lambda j: (0, 0)),
            pl.BlockSpec((_D_IN, _F1), lambda j: (0, jnp.minimum(j, _N1 - 1))),
            pl.BlockSpec((1, _F1), lambda j: (0, jnp.minimum(j, _N1 - 1))),
            pl.BlockSpec((_F, _F2),
                         lambda j: (0, jnp.clip(j - _P1, 0, _N2 - 1))),
            pl.BlockSpec((1, _F2),
                         lambda j: (0, jnp.clip(j - _P1, 0, _N2 - 1))),
            pl.BlockSpec((1, _F2),
                         lambda j: (0, jnp.clip(j - _P1, 0, _N2 - 1))),
            pl.BlockSpec((_B, _F2),
                         lambda j: (0, jnp.clip(j - _P2, 0, _N2 - 1))),
        ],
        out_specs=pl.BlockSpec((_B, _F2),
                               lambda j: (0, jnp.clip(j - _P2, 0, _N2 - 1))),
        out_shape=jax.ShapeDtypeStruct((_B, _F), jnp.float32),
        scratch_shapes=[
            pltpu.VMEM((_B, _F), jnp.float32),
            pltpu.VMEM((_B, _F), jnp.int32),
            pltpu.VMEM((_B, 1), jnp.int32),
        ],
        compiler_params=pltpu.CompilerParams(
            dimension_semantics=("arbitrary",)),
    )(x, W_orig, b_o, W_policy, b_p, is_sel, ground_truth)
    return out
